# Initial kernel scaffold; baseline (speedup 1.0000x reference)
import jax
import jax.numpy as jnp
from jax.experimental import pallas as pl


def kernel(x, codebook):
    raise NotImplementedError("write your pallas kernel here")


# speedup vs baseline: 1.0507x; 1.0507x over previous
import functools

import jax
import jax.numpy as jnp
from jax import lax
from jax.experimental import pallas as pl
from jax.experimental.pallas import tpu as pltpu
from jax.experimental.pallas import tpu_sc as plsc

CB = 8192
D = 256
NT = 8192
TM = 256
NB = NT // TM
CK = 2048
NC = CB // CK


def _argmin_body(cb_ref, xt_ref, out_ref, norm2_ref):
    @pl.when(pl.program_id(0) == 0)
    def _():
        cb = cb_ref[...]
        norm2_ref[...] = jnp.sum(cb * cb, axis=1, keepdims=True)

    xt = xt_ref[...]
    norm1 = jnp.sum(xt * xt, axis=0, keepdims=True)
    xt_bf = xt.astype(jnp.bfloat16)

    av = jnp.full((1, TM), jnp.inf, jnp.float32)
    ai = jnp.zeros((1, TM), jnp.int32)
    for c in range(NC):
        cbc = cb_ref[pl.ds(c * CK, CK), :]
        dot = lax.dot_general(cbc.astype(jnp.bfloat16), xt_bf,
                              (((1,), (0,)), ((), ())),
                              preferred_element_type=jnp.float32)
        n2c = norm2_ref[pl.ds(c * CK, CK), :]
        dist = (norm1 + n2c) - 2.0 * dot
        mv = jnp.min(dist, axis=0, keepdims=True)
        rows = lax.broadcasted_iota(jnp.int32, dist.shape, 0)
        cand = jnp.where(dist == mv, rows, jnp.int32(CB))
        mi = jnp.min(cand, axis=0, keepdims=True) + c * CK
        take = mv < av
        ai = jnp.where(take, mi, ai)
        av = jnp.where(take, mv, av).astype(jnp.bfloat16).astype(jnp.float32)
    out_ref[0, 0, :] = ai[0, :]


def _nearest_indices(codebook, xt):
    return pl.pallas_call(
        _argmin_body,
        grid=(NB,),
        in_specs=[
            pl.BlockSpec((CB, D), lambda i: (0, 0)),
            pl.BlockSpec((D, TM), lambda i: (0, i)),
        ],
        out_specs=pl.BlockSpec((1, 1, TM), lambda i: (i, 0, 0)),
        out_shape=jax.ShapeDtypeStruct((NB, 1, TM), jnp.int32),
        scratch_shapes=[pltpu.VMEM((CB, 1), jnp.float32)],
    )(codebook, xt)


def _gather_rows(codebook, idx_flat):
    mesh = plsc.VectorSubcoreMesh(core_axis_name="core",
                                  subcore_axis_name="subcore")
    idx2 = idx_flat.reshape(1, NT)
    W = 128

    @functools.partial(
        pl.kernel,
        out_type=jax.ShapeDtypeStruct((NT, D), jnp.float32),
        mesh=mesh,
    )
    def k(cb_hbm, i_hbm, o_hbm):
        def body(i_vmem, o_vmem):
            pltpu.sync_copy(cb_hbm.at[i_vmem.at[0]], o_vmem)

        pltpu.emit_pipeline(
            body,
            grid=(NT // W,),
            in_specs=[pl.BlockSpec((1, W), index_map=lambda i: (0, i))],
            out_specs=[pl.BlockSpec((W, D), index_map=lambda i: (i, 0))],
            core_axis_name=("core", "subcore"),
            dimension_semantics=(pltpu.PARALLEL,),
        )(i_hbm, o_hbm)

    return k(codebook, idx2)


def kernel(x, codebook):
    b, s, d = x.shape
    xt = x.reshape(b * s, d).T
    idx = _nearest_indices(codebook, xt).reshape(b * s)
    zq = _gather_rows(codebook, idx)
    return zq.reshape(b, s, d), idx.reshape(b, s)


# speedup vs baseline: 1.5050x; 1.4324x over previous
import functools

import jax
import jax.numpy as jnp
from jax import lax
from jax.experimental import pallas as pl
from jax.experimental.pallas import tpu as pltpu
from jax.experimental.pallas import tpu_sc as plsc

CB = 8192
D = 256
NT = 8192
TM = 1024
NB = NT // TM
CK = 2048
NC = CB // CK


def _argmin_body(cb_ref, xt_ref, out_ref, norm2_ref, cbneg2_ref):
    @pl.when(pl.program_id(0) == 0)
    def _():
        cb = cb_ref[...]
        norm2_ref[...] = jnp.sum(cb * cb, axis=1, keepdims=True)
        cbneg2_ref[...] = (cb * -2.0).astype(jnp.bfloat16)

    xt = xt_ref[...]
    norm1 = jnp.sum(xt * xt, axis=0, keepdims=True)
    xt_bf = xt.astype(jnp.bfloat16)

    av = jnp.full((1, TM), jnp.inf, jnp.float32)
    ai = jnp.zeros((1, TM), jnp.int32)
    for c in range(NC):
        dotn2 = lax.dot_general(cbneg2_ref[pl.ds(c * CK, CK), :], xt_bf,
                                (((1,), (0,)), ((), ())),
                                preferred_element_type=jnp.float32)
        n2c = norm2_ref[pl.ds(c * CK, CK), :]
        dist = (norm1 + n2c) + dotn2
        mv = jnp.min(dist, axis=0, keepdims=True)
        mi = jnp.argmin(dist, axis=0)[None, :].astype(jnp.int32) + c * CK
        take = mv < av
        ai = jnp.where(take, mi, ai)
        av = jnp.where(take, mv, av).astype(jnp.bfloat16).astype(jnp.float32)
    out_ref[0, 0, :] = ai[0, :]


def _nearest_indices(codebook, xt):
    return pl.pallas_call(
        _argmin_body,
        grid=(NB,),
        in_specs=[
            pl.BlockSpec((CB, D), lambda i: (0, 0)),
            pl.BlockSpec((D, TM), lambda i: (0, i)),
        ],
        out_specs=pl.BlockSpec((1, 1, TM), lambda i: (i, 0, 0)),
        out_shape=jax.ShapeDtypeStruct((NB, 1, TM), jnp.int32),
        scratch_shapes=[pltpu.VMEM((CB, 1), jnp.float32),
                        pltpu.VMEM((CB, D), jnp.bfloat16)],
    )(codebook, xt)


def _gather_rows(codebook, idx_flat):
    mesh = plsc.VectorSubcoreMesh(core_axis_name="core",
                                  subcore_axis_name="subcore")
    idx2 = idx_flat.reshape(1, NT)
    W = 128

    @functools.partial(
        pl.kernel,
        out_type=jax.ShapeDtypeStruct((NT, D), jnp.float32),
        mesh=mesh,
    )
    def k(cb_hbm, i_hbm, o_hbm):
        def body(i_vmem, o_vmem):
            pltpu.sync_copy(cb_hbm.at[i_vmem.at[0]], o_vmem)

        pltpu.emit_pipeline(
            body,
            grid=(NT // W,),
            in_specs=[pl.BlockSpec((1, W), index_map=lambda i: (0, i))],
            out_specs=[pl.BlockSpec((W, D), index_map=lambda i: (i, 0))],
            core_axis_name=("core", "subcore"),
            dimension_semantics=(pltpu.PARALLEL,),
        )(i_hbm, o_hbm)

    return k(codebook, idx2)


def kernel(x, codebook):
    b, s, d = x.shape
    xt = x.reshape(b * s, d).T
    idx = _nearest_indices(codebook, xt).reshape(b * s)
    zq = _gather_rows(codebook, idx)
    return zq.reshape(b, s, d), idx.reshape(b, s)
